# Initial kernel scaffold; baseline (speedup 1.0000x reference)
import jax
import jax.numpy as jnp
from jax.experimental import pallas as pl


def kernel(logits, labels, candidate_ids):
    raise NotImplementedError("write your pallas kernel here")


# speedup vs baseline: 1.6991x; 1.6991x over previous
import functools

import jax
import jax.numpy as jnp
import numpy as np
from jax import lax
from jax.experimental import pallas as pl

SMALLEST_FLOAT = float(np.finfo(np.float32).tiny) / 100.0


def _fused_body(logits_ref, labels_ref, cids_ref, out_ref):
    labels = labels_ref[...]
    logits = logits_ref[...]
    cids = cids_ref[...]

    R, N = labels.shape
    rowmax = jnp.max(labels, axis=1, keepdims=True)
    iota = lax.broadcasted_iota(jnp.int32, (R, N), 1)
    masked_idx = jnp.where(labels == rowmax, iota, N)
    idx = jnp.min(masked_idx, axis=1, keepdims=True)
    pos_cid = jnp.sum(jnp.where(iota == idx, cids, 0), axis=1, keepdims=True)
    dup = (pos_cid == cids).astype(jnp.float32)
    out_ref[...] = logits + (dup - labels) * SMALLEST_FLOAT


@jax.jit
def kernel(logits, labels, candidate_ids):
    B, N = logits.shape
    R = 256
    cids2d = candidate_ids.reshape(1, N)
    grid = (B // R,)
    return pl.pallas_call(
        _fused_body,
        grid=grid,
        in_specs=[
            pl.BlockSpec((R, N), lambda i: (i, 0)),
            pl.BlockSpec((R, N), lambda i: (i, 0)),
            pl.BlockSpec((1, N), lambda i: (0, 0)),
        ],
        out_specs=pl.BlockSpec((R, N), lambda i: (i, 0)),
        out_shape=jax.ShapeDtypeStruct((B, N), jnp.float32),
    )(logits, labels, cids2d)


# speedup vs baseline: 1.7650x; 1.0388x over previous
import functools

import jax
import jax.numpy as jnp
import numpy as np
from jax import lax
from jax.experimental import pallas as pl

SMALLEST_FLOAT = float(np.finfo(np.float32).tiny) / 100.0


def _fused_body(logits_ref, labels_ref, cids_ref, out_ref):
    labels = labels_ref[...]
    logits = logits_ref[...]
    cids = cids_ref[...]

    R, N = labels.shape
    rowmax = jnp.max(labels, axis=1, keepdims=True)
    iota = lax.broadcasted_iota(jnp.int32, (R, N), 1)
    masked_idx = jnp.where(labels == rowmax, iota, N)
    idx = jnp.min(masked_idx, axis=1, keepdims=True)
    pos_cid = jnp.sum(jnp.where(iota == idx, cids, 0), axis=1, keepdims=True)
    dup = (pos_cid == cids).astype(jnp.float32)
    out_ref[...] = logits + (dup - labels) * SMALLEST_FLOAT


@jax.jit
def kernel(logits, labels, candidate_ids):
    B, N = logits.shape
    R = 512
    cids2d = candidate_ids.reshape(1, N)
    grid = (B // R,)
    return pl.pallas_call(
        _fused_body,
        grid=grid,
        in_specs=[
            pl.BlockSpec((R, N), lambda i: (i, 0)),
            pl.BlockSpec((R, N), lambda i: (i, 0)),
            pl.BlockSpec((1, N), lambda i: (0, 0)),
        ],
        out_specs=pl.BlockSpec((R, N), lambda i: (i, 0)),
        out_shape=jax.ShapeDtypeStruct((B, N), jnp.float32),
    )(logits, labels, cids2d)
